# initial kernel scaffold (unmeasured)
import jax
import jax.numpy as jnp
from jax import lax
from jax.experimental import pallas as pl
from jax.experimental.pallas import tpu as pltpu

N_DEV = 4
SQ = 2048
SKV = 2048
D_MODEL = 1024
H_PER = 8
DH = 128
BQ = 512
SCALE = 0.08838834764831843
BLK = 64


def _qproj_body(x_ref, wq_ref, q_ref):
    x = x_ref[...].astype(jnp.bfloat16)
    wq = wq_ref[...].astype(jnp.bfloat16)
    q_ref[...] = jnp.dot(
        x, wq, preferred_element_type=jnp.float32
    ).astype(jnp.bfloat16)


def _attn_body(q_ref, k_ref, v_ref, ctx_ref):
    qi = pl.program_id(1)
    q = q_ref[...]
    k = k_ref[:, 0, :]
    s = lax.dot_general(
        q, k, (((1,), (1,)), ((), ())), preferred_element_type=jnp.float32
    ) * SCALE
    row = lax.broadcasted_iota(jnp.int32, s.shape, 0) + qi * BQ
    col = lax.broadcasted_iota(jnp.int32, s.shape, 1)
    s = jnp.where((col // BLK) <= (row // BLK), s, -1e9)
    m = jnp.max(s, axis=1, keepdims=True)
    w = jnp.exp(s - m)
    denom = jnp.sum(w, axis=1, keepdims=True)
    w = (w / denom).astype(jnp.bfloat16)
    ctx_ref[...] = jnp.dot(
        w, v_ref[:, 0, :], preferred_element_type=jnp.float32
    ).astype(jnp.bfloat16)


def _oproj_body(ctx_ref, wo_ref, p_ref):
    ctx = ctx_ref[...]
    wo = wo_ref[...].astype(jnp.bfloat16)
    p_ref[...] = jnp.dot(
        ctx, wo, preferred_element_type=jnp.float32
    ).astype(jnp.bfloat16)


def _allreduce_body(p_ref, out_ref, comm_ref, send_sems, recv_sems):
    my = lax.axis_index("i")
    left = (my + N_DEV - 1) % N_DEV
    right = (my + 1) % N_DEV

    barrier = pltpu.get_barrier_semaphore()
    for nbr in (left, right):
        pl.semaphore_signal(
            barrier, inc=1, device_id=(nbr,),
            device_id_type=pl.DeviceIdType.MESH,
        )
    pl.semaphore_wait(barrier, 2)

    comm_ref[0] = p_ref[...]
    out_ref[...] = p_ref[...].astype(jnp.float32)

    for h in range(N_DEV - 1):
        send_slot = h % 2
        recv_slot = (h + 1) % 2
        rdma = pltpu.make_async_remote_copy(
            src_ref=comm_ref.at[send_slot],
            dst_ref=comm_ref.at[recv_slot],
            send_sem=send_sems.at[send_slot],
            recv_sem=recv_sems.at[recv_slot],
            device_id=(right,),
            device_id_type=pl.DeviceIdType.MESH,
        )
        rdma.start()
        rdma.wait()
        out_ref[...] += comm_ref[recv_slot].astype(jnp.float32)


def kernel(x, Wq, K_ext, V_ext, Wo):
    my = lax.axis_index("i")
    x2 = x.reshape(SQ, D_MODEL)
    K = lax.dynamic_slice_in_dim(
        K_ext.reshape(SKV, 32, DH), my * H_PER, H_PER, axis=1
    ).astype(jnp.bfloat16)
    V = lax.dynamic_slice_in_dim(
        V_ext.reshape(SKV, 32, DH), my * H_PER, H_PER, axis=1
    ).astype(jnp.bfloat16)

    Q = pl.pallas_call(
        _qproj_body,
        out_shape=jax.ShapeDtypeStruct((SQ, D_MODEL), jnp.bfloat16),
        in_specs=[
            pl.BlockSpec(memory_space=pltpu.VMEM),
            pl.BlockSpec(memory_space=pltpu.VMEM),
        ],
        out_specs=pl.BlockSpec(memory_space=pltpu.VMEM),
    )(x2, Wq)

    ctx = pl.pallas_call(
        _attn_body,
        grid=(H_PER, SQ // BQ),
        out_shape=jax.ShapeDtypeStruct((SQ, H_PER * DH), jnp.bfloat16),
        in_specs=[
            pl.BlockSpec((BQ, DH), lambda h, qi: (qi, h)),
            pl.BlockSpec((SKV, 1, DH), lambda h, qi: (0, h, 0)),
            pl.BlockSpec((SKV, 1, DH), lambda h, qi: (0, h, 0)),
        ],
        out_specs=pl.BlockSpec((BQ, DH), lambda h, qi: (qi, h)),
    )(Q, K, V)

    partial = pl.pallas_call(
        _oproj_body,
        out_shape=jax.ShapeDtypeStruct((SQ, D_MODEL), jnp.bfloat16),
        in_specs=[
            pl.BlockSpec(memory_space=pltpu.VMEM),
            pl.BlockSpec(memory_space=pltpu.VMEM),
        ],
        out_specs=pl.BlockSpec(memory_space=pltpu.VMEM),
    )(ctx, Wo)

    out = pl.pallas_call(
        _allreduce_body,
        out_shape=jax.ShapeDtypeStruct((SQ, D_MODEL), jnp.float32),
        in_specs=[pl.BlockSpec(memory_space=pltpu.VMEM)],
        out_specs=pl.BlockSpec(memory_space=pltpu.VMEM),
        scratch_shapes=[
            pltpu.VMEM((2, SQ, D_MODEL), jnp.bfloat16),
            pltpu.SemaphoreType.DMA((2,)),
            pltpu.SemaphoreType.DMA((2,)),
        ],
        compiler_params=pltpu.CompilerParams(collective_id=0),
    )(partial)

    return out.reshape(1, SQ, D_MODEL)


# baseline (device time: 284569 ns/iter reference)
import jax
import jax.numpy as jnp
from jax import lax
from jax.experimental import pallas as pl
from jax.experimental.pallas import tpu as pltpu

N_DEV = 4
SQ = 2048
SKV = 2048
D_MODEL = 1024
H_PER = 8
DH = 128
BQ = 512
SCALE = 0.08838834764831843
BLK = 64


def _qproj_body(x_ref, wq_ref, q_ref):
    x = x_ref[...].astype(jnp.bfloat16)
    wq = wq_ref[...].astype(jnp.bfloat16)
    q_ref[...] = jnp.dot(
        x, wq, preferred_element_type=jnp.float32
    ).astype(jnp.bfloat16)


def _attn_body(q_ref, k_ref, v_ref, ctx_ref):
    qi = pl.program_id(1)
    q = q_ref[...]
    k = k_ref[0]
    s = lax.dot_general(
        q, k, (((1,), (1,)), ((), ())), preferred_element_type=jnp.float32
    ) * SCALE
    row = lax.broadcasted_iota(jnp.int32, s.shape, 0) + qi * BQ
    col = lax.broadcasted_iota(jnp.int32, s.shape, 1)
    s = jnp.where((col // BLK) <= (row // BLK), s, -1e9)
    m = jnp.max(s, axis=1, keepdims=True)
    w = jnp.exp(s - m)
    denom = jnp.sum(w, axis=1, keepdims=True)
    w = (w / denom).astype(jnp.bfloat16)
    ctx_ref[...] = jnp.dot(
        w, v_ref[0], preferred_element_type=jnp.float32
    ).astype(jnp.bfloat16)


def _oproj_body(ctx_ref, wo_ref, p_ref):
    ctx = ctx_ref[...]
    wo = wo_ref[...].astype(jnp.bfloat16)
    p_ref[...] = jnp.dot(
        ctx, wo, preferred_element_type=jnp.float32
    ).astype(jnp.bfloat16)


def _allreduce_body(p_ref, out_ref, comm_ref, send_sems, recv_sems):
    my = lax.axis_index("i")
    left = (my + N_DEV - 1) % N_DEV
    right = (my + 1) % N_DEV

    barrier = pltpu.get_barrier_semaphore()
    for nbr in (left, right):
        pl.semaphore_signal(
            barrier, inc=1, device_id=(nbr,),
            device_id_type=pl.DeviceIdType.MESH,
        )
    pl.semaphore_wait(barrier, 2)

    comm_ref[0] = p_ref[...]
    out_ref[...] = p_ref[...].astype(jnp.float32)

    for h in range(N_DEV - 1):
        send_slot = h % 2
        recv_slot = (h + 1) % 2
        rdma = pltpu.make_async_remote_copy(
            src_ref=comm_ref.at[send_slot],
            dst_ref=comm_ref.at[recv_slot],
            send_sem=send_sems.at[send_slot],
            recv_sem=recv_sems.at[recv_slot],
            device_id=(right,),
            device_id_type=pl.DeviceIdType.MESH,
        )
        rdma.start()
        rdma.wait()
        out_ref[...] += comm_ref[recv_slot].astype(jnp.float32)


def kernel(x, Wq, K_ext, V_ext, Wo):
    my = lax.axis_index("i")
    x2 = x.reshape(SQ, D_MODEL)
    K = lax.dynamic_slice_in_dim(
        K_ext.reshape(SKV, 32, DH), my * H_PER, H_PER, axis=1
    ).astype(jnp.bfloat16).transpose(1, 0, 2)
    V = lax.dynamic_slice_in_dim(
        V_ext.reshape(SKV, 32, DH), my * H_PER, H_PER, axis=1
    ).astype(jnp.bfloat16).transpose(1, 0, 2)

    Q = pl.pallas_call(
        _qproj_body,
        out_shape=jax.ShapeDtypeStruct((SQ, D_MODEL), jnp.bfloat16),
        in_specs=[
            pl.BlockSpec(memory_space=pltpu.VMEM),
            pl.BlockSpec(memory_space=pltpu.VMEM),
        ],
        out_specs=pl.BlockSpec(memory_space=pltpu.VMEM),
    )(x2, Wq)

    ctx = pl.pallas_call(
        _attn_body,
        grid=(H_PER, SQ // BQ),
        out_shape=jax.ShapeDtypeStruct((SQ, H_PER * DH), jnp.bfloat16),
        in_specs=[
            pl.BlockSpec((BQ, DH), lambda h, qi: (qi, h)),
            pl.BlockSpec((1, SKV, DH), lambda h, qi: (h, 0, 0)),
            pl.BlockSpec((1, SKV, DH), lambda h, qi: (h, 0, 0)),
        ],
        out_specs=pl.BlockSpec((BQ, DH), lambda h, qi: (qi, h)),
    )(Q, K, V)

    partial = pl.pallas_call(
        _oproj_body,
        out_shape=jax.ShapeDtypeStruct((SQ, D_MODEL), jnp.bfloat16),
        in_specs=[
            pl.BlockSpec(memory_space=pltpu.VMEM),
            pl.BlockSpec(memory_space=pltpu.VMEM),
        ],
        out_specs=pl.BlockSpec(memory_space=pltpu.VMEM),
    )(ctx, Wo)

    out = pl.pallas_call(
        _allreduce_body,
        out_shape=jax.ShapeDtypeStruct((SQ, D_MODEL), jnp.float32),
        in_specs=[pl.BlockSpec(memory_space=pltpu.VMEM)],
        out_specs=pl.BlockSpec(memory_space=pltpu.VMEM),
        scratch_shapes=[
            pltpu.VMEM((2, SQ, D_MODEL), jnp.bfloat16),
            pltpu.SemaphoreType.DMA((2,)),
            pltpu.SemaphoreType.DMA((2,)),
        ],
        compiler_params=pltpu.CompilerParams(collective_id=0),
    )(partial)

    return out.reshape(1, SQ, D_MODEL)


# device time: 190738 ns/iter; 1.4919x vs baseline; 1.4919x over previous
import jax
import jax.numpy as jnp
from jax import lax
from jax.experimental import pallas as pl
from jax.experimental.pallas import tpu as pltpu

N_DEV = 4
SQ = 2048
SKV = 2048
D_MODEL = 1024
H_PER = 8
DH = 128
BQ = 512
SCALE = 0.08838834764831843
BLK = 64


def _qproj_body(x_ref, wq_ref, q_ref):
    x = x_ref[...].astype(jnp.bfloat16)
    wq = wq_ref[...].astype(jnp.bfloat16)
    q_ref[...] = jnp.dot(
        x, wq, preferred_element_type=jnp.float32
    ).astype(jnp.bfloat16)


def _attn_body(q_ref, k_ref, v_ref, ctx_ref):
    qi = pl.program_id(1)
    q = q_ref[...]
    k = k_ref[0]
    s = lax.dot_general(
        q, k, (((1,), (1,)), ((), ())), preferred_element_type=jnp.float32
    ) * SCALE
    row = lax.broadcasted_iota(jnp.int32, s.shape, 0) + qi * BQ
    col = lax.broadcasted_iota(jnp.int32, s.shape, 1)
    s = jnp.where((col // BLK) <= (row // BLK), s, -1e9)
    m = jnp.max(s, axis=1, keepdims=True)
    w = jnp.exp(s - m)
    denom = jnp.sum(w, axis=1, keepdims=True)
    w = (w / denom).astype(jnp.bfloat16)
    ctx_ref[...] = jnp.dot(
        w, v_ref[0], preferred_element_type=jnp.float32
    ).astype(jnp.bfloat16)


def _oproj_body(ctx_ref, wo_ref, p_ref):
    ctx = ctx_ref[...]
    wo = wo_ref[...].astype(jnp.bfloat16)
    p_ref[...] = jnp.dot(
        ctx, wo, preferred_element_type=jnp.float32
    ).astype(jnp.bfloat16)


HALF = SQ // 2


def _allreduce_body(p_ref, out_ref, sbuf_ref, rbuf1_ref, rbuf2_ref,
                    send_sems, recv_sems):
    my = lax.axis_index("i")
    partner_a = my ^ 1
    partner_b = 3 - my

    barrier = pltpu.get_barrier_semaphore()
    for nbr in (partner_a, partner_b):
        pl.semaphore_signal(
            barrier, inc=1, device_id=(nbr,),
            device_id_type=pl.DeviceIdType.MESH,
        )
    pl.semaphore_wait(barrier, 2)

    r1a = pltpu.make_async_remote_copy(
        src_ref=p_ref.at[pl.ds(0, HALF)],
        dst_ref=rbuf1_ref.at[0],
        send_sem=send_sems.at[0, 0],
        recv_sem=recv_sems.at[0, 0],
        device_id=(partner_a,),
        device_id_type=pl.DeviceIdType.MESH,
    )
    r1b = pltpu.make_async_remote_copy(
        src_ref=p_ref.at[pl.ds(HALF, HALF)],
        dst_ref=rbuf1_ref.at[1],
        send_sem=send_sems.at[0, 1],
        recv_sem=recv_sems.at[0, 1],
        device_id=(partner_b,),
        device_id_type=pl.DeviceIdType.MESH,
    )
    r1a.start()
    r1b.start()
    r1a.wait()
    r1b.wait()

    sbuf_ref[0] = p_ref[pl.ds(0, HALF)] + rbuf1_ref[0]
    sbuf_ref[1] = p_ref[pl.ds(HALF, HALF)] + rbuf1_ref[1]

    r2a = pltpu.make_async_remote_copy(
        src_ref=sbuf_ref.at[0],
        dst_ref=rbuf2_ref.at[0],
        send_sem=send_sems.at[1, 0],
        recv_sem=recv_sems.at[1, 0],
        device_id=(partner_b,),
        device_id_type=pl.DeviceIdType.MESH,
    )
    r2b = pltpu.make_async_remote_copy(
        src_ref=sbuf_ref.at[1],
        dst_ref=rbuf2_ref.at[1],
        send_sem=send_sems.at[1, 1],
        recv_sem=recv_sems.at[1, 1],
        device_id=(partner_a,),
        device_id_type=pl.DeviceIdType.MESH,
    )
    r2a.start()
    r2b.start()
    r2a.wait()
    r2b.wait()

    out_ref[pl.ds(0, HALF)] = (
        sbuf_ref[0].astype(jnp.float32) + rbuf2_ref[0].astype(jnp.float32)
    )
    out_ref[pl.ds(HALF, HALF)] = (
        sbuf_ref[1].astype(jnp.float32) + rbuf2_ref[1].astype(jnp.float32)
    )


def kernel(x, Wq, K_ext, V_ext, Wo):
    my = lax.axis_index("i")
    x2 = x.reshape(SQ, D_MODEL)
    K = lax.dynamic_slice_in_dim(
        K_ext.reshape(SKV, 32, DH), my * H_PER, H_PER, axis=1
    ).astype(jnp.bfloat16).transpose(1, 0, 2)
    V = lax.dynamic_slice_in_dim(
        V_ext.reshape(SKV, 32, DH), my * H_PER, H_PER, axis=1
    ).astype(jnp.bfloat16).transpose(1, 0, 2)

    Q = pl.pallas_call(
        _qproj_body,
        out_shape=jax.ShapeDtypeStruct((SQ, D_MODEL), jnp.bfloat16),
        in_specs=[
            pl.BlockSpec(memory_space=pltpu.VMEM),
            pl.BlockSpec(memory_space=pltpu.VMEM),
        ],
        out_specs=pl.BlockSpec(memory_space=pltpu.VMEM),
    )(x2, Wq)

    ctx = pl.pallas_call(
        _attn_body,
        grid=(H_PER, SQ // BQ),
        out_shape=jax.ShapeDtypeStruct((SQ, H_PER * DH), jnp.bfloat16),
        in_specs=[
            pl.BlockSpec((BQ, DH), lambda h, qi: (qi, h)),
            pl.BlockSpec((1, SKV, DH), lambda h, qi: (h, 0, 0)),
            pl.BlockSpec((1, SKV, DH), lambda h, qi: (h, 0, 0)),
        ],
        out_specs=pl.BlockSpec((BQ, DH), lambda h, qi: (qi, h)),
    )(Q, K, V)

    partial = pl.pallas_call(
        _oproj_body,
        out_shape=jax.ShapeDtypeStruct((SQ, D_MODEL), jnp.bfloat16),
        in_specs=[
            pl.BlockSpec(memory_space=pltpu.VMEM),
            pl.BlockSpec(memory_space=pltpu.VMEM),
        ],
        out_specs=pl.BlockSpec(memory_space=pltpu.VMEM),
    )(ctx, Wo)

    out = pl.pallas_call(
        _allreduce_body,
        out_shape=jax.ShapeDtypeStruct((SQ, D_MODEL), jnp.float32),
        in_specs=[pl.BlockSpec(memory_space=pltpu.VMEM)],
        out_specs=pl.BlockSpec(memory_space=pltpu.VMEM),
        scratch_shapes=[
            pltpu.VMEM((2, HALF, D_MODEL), jnp.bfloat16),
            pltpu.VMEM((2, HALF, D_MODEL), jnp.bfloat16),
            pltpu.VMEM((2, HALF, D_MODEL), jnp.bfloat16),
            pltpu.SemaphoreType.DMA((2, 2)),
            pltpu.SemaphoreType.DMA((2, 2)),
        ],
        compiler_params=pltpu.CompilerParams(collective_id=0),
    )(partial)

    return out.reshape(1, SQ, D_MODEL)


# device time: 167121 ns/iter; 1.7028x vs baseline; 1.1413x over previous
import jax
import jax.numpy as jnp
from jax import lax
from jax.experimental import pallas as pl
from jax.experimental.pallas import tpu as pltpu

N_DEV = 4
SQ = 2048
SKV = 2048
D_MODEL = 1024
H_PER = 8
DH = 128
BQ = 512
SCALE = 0.08838834764831843
BLK = 64


def _qproj_body(x_ref, wq_ref, q_ref):
    x = x_ref[...].astype(jnp.bfloat16)
    wq = wq_ref[...].astype(jnp.bfloat16)
    q_ref[...] = jnp.dot(
        x, wq, preferred_element_type=jnp.float32
    ).astype(jnp.bfloat16)


BKV = 512


def _attn_body(q_ref, k_ref, v_ref, ctx_ref):
    qi = pl.program_id(1)
    q = q_ref[...]

    def step(j, carry):
        acc, m, l = carry
        k = k_ref[0, pl.ds(j * BKV, BKV), :]
        s = lax.dot_general(
            q, k, (((1,), (1,)), ((), ())),
            preferred_element_type=jnp.float32,
        ) * SCALE
        rowg = lax.broadcasted_iota(jnp.int32, (BQ, BKV), 0) + qi * BQ
        colg = lax.broadcasted_iota(jnp.int32, (BQ, BKV), 1) + j * BKV
        s = jnp.where((colg // BLK) <= (rowg // BLK), s, -1e9)
        m_new = jnp.maximum(m, jnp.max(s, axis=1, keepdims=True))
        p = jnp.exp(s - m_new)
        alpha = jnp.exp(m - m_new)
        l_new = l * alpha + jnp.sum(p, axis=1, keepdims=True)
        v = v_ref[0, pl.ds(j * BKV, BKV), :]
        acc_new = acc * alpha + jnp.dot(
            p.astype(jnp.bfloat16), v, preferred_element_type=jnp.float32
        )
        return acc_new, m_new, l_new

    init = (
        jnp.zeros((BQ, DH), jnp.float32),
        jnp.full((BQ, 1), -1e30, jnp.float32),
        jnp.zeros((BQ, 1), jnp.float32),
    )
    acc, m, l = lax.fori_loop(0, qi + 1, step, init)
    ctx_ref[...] = (acc / l).astype(jnp.bfloat16)


def _oproj_body(ctx_ref, wo_ref, p_ref):
    ctx = ctx_ref[...]
    wo = wo_ref[...].astype(jnp.bfloat16)
    p_ref[...] = jnp.dot(
        ctx, wo, preferred_element_type=jnp.float32
    ).astype(jnp.bfloat16)


HALF = SQ // 2


def _allreduce_body(p_ref, out_ref, sbuf_ref, rbuf1_ref, rbuf2_ref,
                    send_sems, recv_sems):
    my = lax.axis_index("i")
    partner_a = my ^ 1
    partner_b = 3 - my

    barrier = pltpu.get_barrier_semaphore()
    for nbr in (partner_a, partner_b):
        pl.semaphore_signal(
            barrier, inc=1, device_id=(nbr,),
            device_id_type=pl.DeviceIdType.MESH,
        )
    pl.semaphore_wait(barrier, 2)

    r1a = pltpu.make_async_remote_copy(
        src_ref=p_ref.at[pl.ds(0, HALF)],
        dst_ref=rbuf1_ref.at[0],
        send_sem=send_sems.at[0, 0],
        recv_sem=recv_sems.at[0, 0],
        device_id=(partner_a,),
        device_id_type=pl.DeviceIdType.MESH,
    )
    r1b = pltpu.make_async_remote_copy(
        src_ref=p_ref.at[pl.ds(HALF, HALF)],
        dst_ref=rbuf1_ref.at[1],
        send_sem=send_sems.at[0, 1],
        recv_sem=recv_sems.at[0, 1],
        device_id=(partner_b,),
        device_id_type=pl.DeviceIdType.MESH,
    )
    r1a.start()
    r1b.start()
    r1a.wait()
    r1b.wait()

    sbuf_ref[0] = p_ref[pl.ds(0, HALF)] + rbuf1_ref[0]
    sbuf_ref[1] = p_ref[pl.ds(HALF, HALF)] + rbuf1_ref[1]

    r2a = pltpu.make_async_remote_copy(
        src_ref=sbuf_ref.at[0],
        dst_ref=rbuf2_ref.at[0],
        send_sem=send_sems.at[1, 0],
        recv_sem=recv_sems.at[1, 0],
        device_id=(partner_b,),
        device_id_type=pl.DeviceIdType.MESH,
    )
    r2b = pltpu.make_async_remote_copy(
        src_ref=sbuf_ref.at[1],
        dst_ref=rbuf2_ref.at[1],
        send_sem=send_sems.at[1, 1],
        recv_sem=recv_sems.at[1, 1],
        device_id=(partner_a,),
        device_id_type=pl.DeviceIdType.MESH,
    )
    r2a.start()
    r2b.start()
    r2a.wait()
    r2b.wait()

    out_ref[pl.ds(0, HALF)] = (
        sbuf_ref[0].astype(jnp.float32) + rbuf2_ref[0].astype(jnp.float32)
    )
    out_ref[pl.ds(HALF, HALF)] = (
        sbuf_ref[1].astype(jnp.float32) + rbuf2_ref[1].astype(jnp.float32)
    )


def kernel(x, Wq, K_ext, V_ext, Wo):
    my = lax.axis_index("i")
    x2 = x.reshape(SQ, D_MODEL)
    K = lax.dynamic_slice_in_dim(
        K_ext.reshape(SKV, 32, DH), my * H_PER, H_PER, axis=1
    ).astype(jnp.bfloat16).transpose(1, 0, 2)
    V = lax.dynamic_slice_in_dim(
        V_ext.reshape(SKV, 32, DH), my * H_PER, H_PER, axis=1
    ).astype(jnp.bfloat16).transpose(1, 0, 2)

    Q = pl.pallas_call(
        _qproj_body,
        out_shape=jax.ShapeDtypeStruct((SQ, D_MODEL), jnp.bfloat16),
        in_specs=[
            pl.BlockSpec(memory_space=pltpu.VMEM),
            pl.BlockSpec(memory_space=pltpu.VMEM),
        ],
        out_specs=pl.BlockSpec(memory_space=pltpu.VMEM),
    )(x2, Wq)

    ctx = pl.pallas_call(
        _attn_body,
        grid=(H_PER, SQ // BQ),
        out_shape=jax.ShapeDtypeStruct((SQ, H_PER * DH), jnp.bfloat16),
        in_specs=[
            pl.BlockSpec((BQ, DH), lambda h, qi: (qi, h)),
            pl.BlockSpec((1, SKV, DH), lambda h, qi: (h, 0, 0)),
            pl.BlockSpec((1, SKV, DH), lambda h, qi: (h, 0, 0)),
        ],
        out_specs=pl.BlockSpec((BQ, DH), lambda h, qi: (qi, h)),
    )(Q, K, V)

    partial = pl.pallas_call(
        _oproj_body,
        out_shape=jax.ShapeDtypeStruct((SQ, D_MODEL), jnp.bfloat16),
        in_specs=[
            pl.BlockSpec(memory_space=pltpu.VMEM),
            pl.BlockSpec(memory_space=pltpu.VMEM),
        ],
        out_specs=pl.BlockSpec(memory_space=pltpu.VMEM),
    )(ctx, Wo)

    out = pl.pallas_call(
        _allreduce_body,
        out_shape=jax.ShapeDtypeStruct((SQ, D_MODEL), jnp.float32),
        in_specs=[pl.BlockSpec(memory_space=pltpu.VMEM)],
        out_specs=pl.BlockSpec(memory_space=pltpu.VMEM),
        scratch_shapes=[
            pltpu.VMEM((2, HALF, D_MODEL), jnp.bfloat16),
            pltpu.VMEM((2, HALF, D_MODEL), jnp.bfloat16),
            pltpu.VMEM((2, HALF, D_MODEL), jnp.bfloat16),
            pltpu.SemaphoreType.DMA((2, 2)),
            pltpu.SemaphoreType.DMA((2, 2)),
        ],
        compiler_params=pltpu.CompilerParams(collective_id=0),
    )(partial)

    return out.reshape(1, SQ, D_MODEL)
